# baseline (speedup 1.0000x reference)
import functools

import jax
import jax.numpy as jnp
from jax import lax
from jax.experimental import pallas as pl
from jax.experimental.pallas import tpu as pltpu
from jax.experimental.pallas import tpu_sc as plsc

B, S, D = 1024, 64, 1024
NC, NS, L = 2, 16, 16
NW = NC * NS
S_PER_W = S // NW
NB = 4
NV = D // L
NCH = B // NB
NIN = 8
NOUT = 4
PREF = 6


def _sc_kernel(src_hbm, pos_hbm, out_hbm, pos_v, *bufs_and_sems):
    ins = bufs_and_sems[:NIN]
    outs = bufs_and_sems[NIN:NIN + NOUT]
    gsems = bufs_and_sems[NIN + NOUT:2 * NIN + NOUT]
    ssems = bufs_and_sems[2 * NIN + NOUT:]

    wid = lax.axis_index("s") * NC + lax.axis_index("c")
    s0 = wid * S_PER_W
    pltpu.sync_copy(pos_hbm.at[pl.ds(s0, S_PER_W)], pos_v)

    def gather(ci, bi):
        return pltpu.make_async_copy(
            src_hbm.at[pl.ds(ci * NB, NB), pl.ds(s0, S_PER_W)], ins[bi], gsems[bi])

    def scatter(ci, bo):
        return pltpu.make_async_copy(
            outs[bo], out_hbm.at[pl.ds(ci * NB, NB), pl.ds(s0, S_PER_W)], ssems[bo])

    def compute(src_v, dst_v):
        @plsc.parallel_loop(0, NV, step=1, unroll=2)
        def _(j):
            off = j * L
            for p in range(S_PER_W):
                pv = pos_v[p, pl.ds(off, L)]
                for b in range(NB):
                    dst_v[b, p, pl.ds(off, L)] = src_v[b, p, pl.ds(off, L)] + pv

    for k in range(PREF):
        gather(k, k).start()

    def body(g, carry):
        for u in range(NIN):
            ci = g * NIN + u
            bo = u % NOUT

            @pl.when(ci + PREF < NCH)
            def _():
                gather(ci + PREF, (u + PREF) % NIN).start()

            gather(ci, u).wait()

            @pl.when(ci >= NOUT)
            def _():
                scatter(ci - NOUT, bo).wait()

            compute(ins[u], outs[bo])
            scatter(ci, bo).start()
        return carry

    lax.fori_loop(0, NCH // NIN, body, 0)
    for k in range(NOUT):
        scatter(NCH - NOUT + k, k).wait()


def kernel(src, pos_embed):
    mesh = plsc.VectorSubcoreMesh(core_axis_name="c", subcore_axis_name="s")
    scratch = (
        [pltpu.VMEM((S_PER_W, D), jnp.float32)]
        + [pltpu.VMEM((NB, S_PER_W, D), jnp.float32) for _ in range(NIN + NOUT)]
        + [pltpu.SemaphoreType.DMA for _ in range(NIN + NOUT)]
    )
    f = functools.partial(
        pl.kernel,
        mesh=mesh,
        out_type=jax.ShapeDtypeStruct((B, S, D), jnp.float32),
        scratch_types=scratch,
    )(_sc_kernel)
    return f(src, pos_embed)


# speedup vs baseline: 1.0078x; 1.0078x over previous
import functools

import jax
import jax.numpy as jnp
from jax import lax
from jax.experimental import pallas as pl
from jax.experimental.pallas import tpu as pltpu
from jax.experimental.pallas import tpu_sc as plsc

B, S, D = 1024, 64, 1024
NC, NS, L = 2, 16, 16
NW = NC * NS
S_PER_W = S // NW
NB = 8
NV = D // L
NCH = B // NB
NIN = 2
NOUT = 2
PREF = 1


def _sc_kernel(src_hbm, pos_hbm, out_hbm, pos_v, *bufs_and_sems):
    ins = bufs_and_sems[:NIN]
    outs = bufs_and_sems[NIN:NIN + NOUT]
    gsems = bufs_and_sems[NIN + NOUT:2 * NIN + NOUT]
    ssems = bufs_and_sems[2 * NIN + NOUT:]

    wid = lax.axis_index("s") * NC + lax.axis_index("c")
    s0 = wid * S_PER_W
    pltpu.sync_copy(pos_hbm.at[pl.ds(s0, S_PER_W)], pos_v)

    def gather(ci, bi):
        return pltpu.make_async_copy(
            src_hbm.at[pl.ds(ci * NB, NB), pl.ds(s0, S_PER_W)], ins[bi], gsems[bi])

    def scatter(ci, bo):
        return pltpu.make_async_copy(
            outs[bo], out_hbm.at[pl.ds(ci * NB, NB), pl.ds(s0, S_PER_W)], ssems[bo])

    def compute(src_v, dst_v):
        @plsc.parallel_loop(0, NV, step=1, unroll=2)
        def _(j):
            off = j * L
            for p in range(S_PER_W):
                pv = pos_v[p, pl.ds(off, L)]
                for b in range(NB):
                    dst_v[b, p, pl.ds(off, L)] = src_v[b, p, pl.ds(off, L)] + pv

    for k in range(PREF):
        gather(k, k).start()

    def body(g, carry):
        for u in range(NIN):
            ci = g * NIN + u
            bo = u % NOUT

            @pl.when(ci + PREF < NCH)
            def _():
                gather(ci + PREF, (u + PREF) % NIN).start()

            gather(ci, u).wait()

            @pl.when(ci >= NOUT)
            def _():
                scatter(ci - NOUT, bo).wait()

            compute(ins[u], outs[bo])
            scatter(ci, bo).start()
        return carry

    lax.fori_loop(0, NCH // NIN, body, 0)
    for k in range(NOUT):
        scatter(NCH - NOUT + k, k).wait()


def kernel(src, pos_embed):
    mesh = plsc.VectorSubcoreMesh(core_axis_name="c", subcore_axis_name="s")
    scratch = (
        [pltpu.VMEM((S_PER_W, D), jnp.float32)]
        + [pltpu.VMEM((NB, S_PER_W, D), jnp.float32) for _ in range(NIN + NOUT)]
        + [pltpu.SemaphoreType.DMA for _ in range(NIN + NOUT)]
    )
    f = functools.partial(
        pl.kernel,
        mesh=mesh,
        out_type=jax.ShapeDtypeStruct((B, S, D), jnp.float32),
        scratch_types=scratch,
    )(_sc_kernel)
    return f(src, pos_embed)
